# Initial kernel scaffold; baseline (speedup 1.0000x reference)
import jax
import jax.numpy as jnp
from jax import lax
from jax.experimental import pallas as pl

N = 24
E = 128


def _elu(v):
    return jnp.where(v > 0, v, jnp.expm1(v))


def _dense_body(src_ref, dst_ref, x_ref, W1_ref, b1_ref, W2_ref, b2_ref,
                W3_ref, b3_ref, fw_ref, fb_ref, f2w_ref, f2b_ref, o_ref):
    src = src_ref[...]
    dst = dst_ref[...]
    S1h = (src == lax.broadcasted_iota(jnp.int32, (E, N), 1)).astype(jnp.float32)
    D1hT = (dst == lax.broadcasted_iota(jnp.int32, (N, E), 0)).astype(jnp.float32)
    deg = jnp.sum(S1h, axis=0)
    pos = deg > 0
    dis = jnp.where(pos, lax.rsqrt(jnp.where(pos, deg, 1.0)), 0.0)
    A = jnp.dot(D1hT, S1h, preferred_element_type=jnp.float32)
    M = -(dis[:, None] * dis[None, :]) * A
    diagv = jnp.where(pos, 0.0, -1.0)
    eye = (lax.broadcasted_iota(jnp.int32, (N, N), 0)
           == lax.broadcasted_iota(jnp.int32, (N, N), 1))
    L = M + jnp.where(eye, diagv[None, :], 0.0)

    def lap(v):
        return jnp.dot(L, v, preferred_element_type=jnp.float32)

    def cheb(h, W_ref, b_ref, K):
        out = jnp.dot(h, W_ref[0], preferred_element_type=jnp.float32)
        Tx0 = h
        Tx1 = lap(h)
        out = out + jnp.dot(Tx1, W_ref[1], preferred_element_type=jnp.float32)
        for k in range(2, K):
            Tx2 = 2.0 * lap(Tx1) - Tx0
            out = out + jnp.dot(Tx2, W_ref[k], preferred_element_type=jnp.float32)
            Tx0, Tx1 = Tx1, Tx2
        return out + b_ref[...][None, :]

    h = _elu(cheb(x_ref[...], W1_ref, b1_ref, 3))
    h = _elu(cheb(h, W2_ref, b2_ref, 3))
    h = _elu(cheb(h, W3_ref, b3_ref, 5))

    prod = h[:, :, None] * fw_ref[...]
    z = jnp.sum(jnp.sum(prod, axis=0), axis=0)
    z = z + fb_ref[...]
    z2 = jnp.dot(z.reshape(1, 128), f2w_ref[...],
                 preferred_element_type=jnp.float32) + f2b_ref[...][None, :]
    m = jnp.max(z2, axis=1, keepdims=True)
    s = z2 - m
    lse = jnp.log(jnp.sum(jnp.exp(s), axis=1, keepdims=True))
    o_ref[...] = s - lse


def kernel(x, edge_index, W1, b1, W2, b2, W3, b3, fc1_w, fc1_b, fc2_w, fc2_b):
    src = edge_index[0].reshape(E, 1)
    dst = edge_index[1].reshape(1, E)
    fw = fc1_w.reshape(N, 32, 128)
    return pl.pallas_call(
        _dense_body,
        out_shape=jax.ShapeDtypeStruct((1, 2), jnp.float32),
    )(src, dst, x, W1, b1, W2, b2, W3, b3, fw, fc1_b, fc2_w, fc2_b)


# speedup vs baseline: 23.0819x; 23.0819x over previous
import jax
import jax.numpy as jnp
from jax import lax
from jax.experimental import pallas as pl

N = 24
E = 128


def _elu(v):
    return jnp.where(v > 0, v, jnp.exp(v) - 1.0)


def _dense_body(src_ref, dst_ref, x_ref, W1_ref, b1_ref, W2_ref, b2_ref,
                W3_ref, b3_ref, fw_ref, fb_ref, f2w_ref, f2b_ref, o_ref):
    src = src_ref[...]
    dst = dst_ref[...]
    S1h = (src == lax.broadcasted_iota(jnp.int32, (E, N), 1)).astype(jnp.float32)
    D1hT = (dst == lax.broadcasted_iota(jnp.int32, (N, E), 0)).astype(jnp.float32)
    deg = jnp.sum(S1h, axis=0)
    pos = deg > 0
    dis = jnp.where(pos, lax.rsqrt(jnp.where(pos, deg, 1.0)), 0.0)
    A = jnp.dot(D1hT, S1h, preferred_element_type=jnp.float32)
    M = -(dis[:, None] * dis[None, :]) * A
    diagv = jnp.where(pos, 0.0, -1.0)
    eye = (lax.broadcasted_iota(jnp.int32, (N, N), 0)
           == lax.broadcasted_iota(jnp.int32, (N, N), 1))
    L = M + jnp.where(eye, diagv[None, :], 0.0)

    def lap(v):
        return jnp.dot(L, v, preferred_element_type=jnp.float32)

    def cheb(h, W_ref, b_ref, K):
        out = jnp.dot(h, W_ref[0], preferred_element_type=jnp.float32)
        Tx0 = h
        Tx1 = lap(h)
        out = out + jnp.dot(Tx1, W_ref[1], preferred_element_type=jnp.float32)
        for k in range(2, K):
            Tx2 = 2.0 * lap(Tx1) - Tx0
            out = out + jnp.dot(Tx2, W_ref[k], preferred_element_type=jnp.float32)
            Tx0, Tx1 = Tx1, Tx2
        return out + b_ref[...][None, :]

    h = _elu(cheb(x_ref[...], W1_ref, b1_ref, 3))
    h = _elu(cheb(h, W2_ref, b2_ref, 3))
    h = _elu(cheb(h, W3_ref, b3_ref, 5))

    prod = h[:, :, None] * fw_ref[...]
    z = jnp.sum(jnp.sum(prod, axis=0), axis=0)
    z = z + fb_ref[...]
    z2 = jnp.dot(z.reshape(1, 128), f2w_ref[...],
                 preferred_element_type=jnp.float32) + f2b_ref[...][None, :]
    m = jnp.max(z2, axis=1, keepdims=True)
    s = z2 - m
    lse = jnp.log(jnp.sum(jnp.exp(s), axis=1, keepdims=True))
    o_ref[...] = s - lse


def kernel(x, edge_index, W1, b1, W2, b2, W3, b3, fc1_w, fc1_b, fc2_w, fc2_b):
    src = edge_index[0].reshape(E, 1)
    dst = edge_index[1].reshape(1, E)
    fw = fc1_w.reshape(N, 32, 128)
    return pl.pallas_call(
        _dense_body,
        out_shape=jax.ShapeDtypeStruct((1, 2), jnp.float32),
    )(src, dst, x, W1, b1, W2, b2, W3, b3, fw, fc1_b, fc2_w, fc2_b)
